# Initial kernel scaffold; baseline (speedup 1.0000x reference)
import functools

import jax
import jax.numpy as jnp
from jax import lax
from jax.experimental import pallas as pl
from jax.experimental.pallas import tpu as pltpu
from jax.experimental.pallas import tpu_sc as plsc

_LANES = 16
_TABLE = 16384
_MAGIC_F = jnp.float32(12582912.0)
_MAGIC_I = jnp.int32(0x4B400000)


def _make_sc_call(n):
  info = plsc.get_sparse_core_info()
  nc, ns = info.num_cores, info.num_subcores
  nw = nc * ns
  per_w = n // nw
  ch = 16384
  n_ch = per_w // ch
  mesh = plsc.VectorSubcoreMesh(core_axis_name="c", subcore_axis_name="s")

  @functools.partial(
      pl.kernel,
      mesh=mesh,
      out_type=jax.ShapeDtypeStruct((n,), jnp.float32),
      scratch_types=[
          pltpu.VMEM((_TABLE,), jnp.float32),
          pltpu.VMEM((ch,), jnp.float32),
          pltpu.VMEM((ch,), jnp.float32),
      ],
  )
  def sc_gelu(x_hbm, lut_hbm, out_hbm, lut_v, xbuf, obuf):
    wid = lax.axis_index("s") * nc + lax.axis_index("c")
    pltpu.sync_copy(lut_hbm, lut_v)

    def chunk_body(g, carry):
      base = wid * per_w + g * ch
      pltpu.sync_copy(x_hbm.at[pl.ds(base, ch)], xbuf)

      def vec_body(i, c2):
        off = i * _LANES
        xv = xbuf[pl.ds(off, _LANES)]
        s = xv * jnp.float32(65536.0) + _MAGIC_F
        xi = plsc.bitcast(s, jnp.int32) - _MAGIC_I
        y = lax.shift_right_arithmetic(xi, 4)
        a = jnp.maximum(y, -y)
        c = jnp.minimum(a, jnp.int32(_TABLE - 1))
        lv = plsc.load_gather(lut_v, [c])
        res = jnp.where(xi >= 0, xv, jnp.float32(0.0)) - lv
        obuf[pl.ds(off, _LANES)] = res
        return c2

      lax.fori_loop(0, ch // _LANES, vec_body, 0)
      pltpu.sync_copy(obuf, out_hbm.at[pl.ds(base, ch)])
      return carry

    lax.fori_loop(0, n_ch, chunk_body, 0)

  return sc_gelu


def kernel(x, lut):
  n = x.size
  sc_gelu = _make_sc_call(n)
  out = sc_gelu(x.reshape(n), lut)
  return out.reshape(x.shape)


# speedup vs baseline: 421.6343x; 421.6343x over previous
import functools

import jax
import jax.numpy as jnp
from jax import lax
from jax.experimental import pallas as pl
from jax.experimental.pallas import tpu as pltpu
from jax.experimental.pallas import tpu_sc as plsc

_LANES = 16
_TABLE = 16384
_MAGIC_F = 12582912.0
_MAGIC_I = 0x4B400000


def _make_sc_call(n):
  info = plsc.get_sparse_core_info()
  nc, ns = info.num_cores, info.num_subcores
  nw = nc * ns
  per_w = n // nw
  ch = 16384
  n_ch = per_w // ch
  mesh = plsc.VectorSubcoreMesh(core_axis_name="c", subcore_axis_name="s")

  @functools.partial(
      pl.kernel,
      mesh=mesh,
      compiler_params=pltpu.CompilerParams(needs_layout_passes=False),
      out_type=jax.ShapeDtypeStruct((n,), jnp.float32),
      scratch_types=[
          pltpu.VMEM((_TABLE,), jnp.float32),
          pltpu.VMEM((ch,), jnp.float32),
          pltpu.VMEM((ch,), jnp.float32),
      ],
  )
  def sc_gelu(x_hbm, lut_hbm, out_hbm, lut_v, xbuf, obuf):
    wid = lax.axis_index("s") * nc + lax.axis_index("c")
    pltpu.sync_copy(lut_hbm, lut_v)

    def chunk_body(g, carry):
      base = wid * per_w + g * ch
      pltpu.sync_copy(x_hbm.at[pl.ds(base, ch)], xbuf)

      def vec_body(i, c2):
        off = i * _LANES
        xv = xbuf[pl.ds(off, _LANES)]
        s = xv * jnp.float32(65536.0) + jnp.float32(_MAGIC_F)
        xi = (s - jnp.float32(_MAGIC_F)).astype(jnp.int32)
        y = lax.shift_right_arithmetic(xi, 4)
        a = jnp.maximum(y, -y)
        c = jnp.minimum(a, jnp.int32(_TABLE - 1))
        lv = plsc.load_gather(lut_v, [c])
        res = jnp.where(xi >= 0, xv, jnp.float32(0.0)) - lv
        obuf[pl.ds(off, _LANES)] = res
        return c2

      lax.fori_loop(0, ch // _LANES, vec_body, 0)
      pltpu.sync_copy(obuf, out_hbm.at[pl.ds(base, ch)])
      return carry

    lax.fori_loop(0, n_ch, chunk_body, 0)

  return sc_gelu


def kernel(x, lut):
  n = x.size
  sc_gelu = _make_sc_call(n)
  out = sc_gelu(x.reshape(n), lut)
  return out.reshape(x.shape)


# speedup vs baseline: 790.9178x; 1.8758x over previous
import functools

import jax
import jax.numpy as jnp
from jax import lax
from jax.experimental import pallas as pl
from jax.experimental.pallas import tpu as pltpu
from jax.experimental.pallas import tpu_sc as plsc

_LANES = 16
_TABLE = 16384
_MAGIC_F = 12582912.0
_CH = 16384
_NBUF = 2


def _make_sc_call(n):
  info = plsc.get_sparse_core_info()
  nc, ns = info.num_cores, info.num_subcores
  nw = nc * ns
  per_w = n // nw
  n_ch = per_w // _CH
  mesh = plsc.VectorSubcoreMesh(core_axis_name="c", subcore_axis_name="s")

  @functools.partial(
      pl.kernel,
      mesh=mesh,
      compiler_params=pltpu.CompilerParams(needs_layout_passes=False),
      out_type=jax.ShapeDtypeStruct((n,), jnp.float32),
      scratch_types=[
          pltpu.VMEM((_TABLE,), jnp.float32),
          [pltpu.VMEM((_CH,), jnp.float32) for _ in range(_NBUF)],
          [pltpu.VMEM((_CH,), jnp.float32) for _ in range(_NBUF)],
          [pltpu.SemaphoreType.DMA for _ in range(_NBUF)],
          [pltpu.SemaphoreType.DMA for _ in range(_NBUF)],
      ],
  )
  def sc_gelu(x_hbm, lut_hbm, out_hbm, lut_v, xbufs, obufs, in_sems,
              out_sems):
    wid = lax.axis_index("s") * nc + lax.axis_index("c")
    base_w = wid * per_w
    pltpu.sync_copy(lut_hbm, lut_v)

    for b in range(_NBUF):
      pltpu.async_copy(
          x_hbm.at[pl.ds(base_w + b * _CH, _CH)], xbufs[b], in_sems[b])

    def chunk_group(gp, carry):
      for b in range(_NBUF):
        g = gp * _NBUF + b
        base = base_w + g * _CH
        pltpu.make_async_copy(
            x_hbm.at[pl.ds(base, _CH)], xbufs[b], in_sems[b]).wait()
        @pl.when(g >= _NBUF)
        def _():
          pltpu.make_async_copy(
              obufs[b], out_hbm.at[pl.ds(base - _NBUF * _CH, _CH)],
              out_sems[b]).wait()

        xbuf, obuf = xbufs[b], obufs[b]

        @plsc.parallel_loop(0, _CH // _LANES, unroll=8)
        def _(i):
          off = i * _LANES
          xv = xbuf[pl.ds(off, _LANES)]
          s = xv * jnp.float32(65536.0) + jnp.float32(_MAGIC_F)
          xi = (s - jnp.float32(_MAGIC_F)).astype(jnp.int32)
          y = lax.shift_right_arithmetic(xi, 4)
          a = jnp.maximum(y, -y)
          c = jnp.minimum(a, jnp.int32(_TABLE - 1))
          lv = plsc.load_gather(lut_v, [c])
          obuf[pl.ds(off, _LANES)] = jnp.where(
              xi >= 0, xv, jnp.float32(0.0)) - lv

        pltpu.async_copy(obuf, out_hbm.at[pl.ds(base, _CH)], out_sems[b])

        @pl.when(g + _NBUF < n_ch)
        def _():
          pltpu.async_copy(
              x_hbm.at[pl.ds(base + _NBUF * _CH, _CH)], xbufs[b],
              in_sems[b])

      return carry

    lax.fori_loop(0, n_ch // _NBUF, chunk_group, 0)

    for b in range(_NBUF):
      g = n_ch - _NBUF + b
      pltpu.make_async_copy(
          obufs[b], out_hbm.at[pl.ds(base_w + g * _CH, _CH)],
          out_sems[b]).wait()

  return sc_gelu


def kernel(x, lut):
  n = x.size
  sc_gelu = _make_sc_call(n)
  out = sc_gelu(x.reshape(n), lut)
  return out.reshape(x.shape)


# speedup vs baseline: 842.8998x; 1.0657x over previous
import functools

import jax
import jax.numpy as jnp
from jax import lax
from jax.experimental import pallas as pl
from jax.experimental.pallas import tpu as pltpu
from jax.experimental.pallas import tpu_sc as plsc

_LANES = 16
_TABLE = 16384
_MAGIC_F = 12582912.0
_MAGIC_I = 0x4B400000
_CH = 16384
_NBUF = 2


def _make_sc_call(n):
  info = plsc.get_sparse_core_info()
  nc, ns = info.num_cores, info.num_subcores
  nw = nc * ns
  per_w = n // nw
  n_ch = per_w // _CH
  mesh = plsc.VectorSubcoreMesh(core_axis_name="c", subcore_axis_name="s")

  @functools.partial(
      pl.kernel,
      mesh=mesh,
      compiler_params=pltpu.CompilerParams(needs_layout_passes=False),
      out_type=jax.ShapeDtypeStruct((n,), jnp.float32),
      scratch_types=[
          pltpu.VMEM((_TABLE,), jnp.float32),
          [pltpu.VMEM((_CH,), jnp.float32) for _ in range(_NBUF)],
          [pltpu.VMEM((_CH,), jnp.float32) for _ in range(_NBUF)],
          [pltpu.SemaphoreType.DMA for _ in range(_NBUF)],
          [pltpu.SemaphoreType.DMA for _ in range(_NBUF)],
      ],
  )
  def sc_gelu(x_hbm, lut_hbm, out_hbm, lut_v, xbufs, obufs, in_sems,
              out_sems):
    wid = lax.axis_index("s") * nc + lax.axis_index("c")
    base_w = wid * per_w
    pltpu.sync_copy(lut_hbm, lut_v)

    for b in range(_NBUF):
      pltpu.async_copy(
          x_hbm.at[pl.ds(base_w + b * _CH, _CH)], xbufs[b], in_sems[b])

    def chunk_group(gp, carry):
      for b in range(_NBUF):
        g = gp * _NBUF + b
        base = base_w + g * _CH
        pltpu.make_async_copy(
            x_hbm.at[pl.ds(base, _CH)], xbufs[b], in_sems[b]).wait()
        @pl.when(g >= _NBUF)
        def _():
          pltpu.make_async_copy(
              obufs[b], out_hbm.at[pl.ds(base - _NBUF * _CH, _CH)],
              out_sems[b]).wait()

        xbuf, obuf = xbufs[b], obufs[b]

        @plsc.parallel_loop(0, _CH // _LANES, unroll=8)
        def _(i):
          off = i * _LANES
          xv = xbuf[pl.ds(off, _LANES)]
          s = xv * jnp.float32(65536.0) + jnp.float32(_MAGIC_F)
          xi = plsc.bitcast(s, jnp.int32) - jnp.int32(_MAGIC_I)
          y = lax.shift_right_arithmetic(xi, 4)
          a = jnp.maximum(y, -y)
          c = jnp.minimum(a, jnp.int32(_TABLE - 1))
          lv = plsc.load_gather(lut_v, [c])
          obuf[pl.ds(off, _LANES)] = jnp.where(
              xi >= 0, xv, jnp.float32(0.0)) - lv

        pltpu.async_copy(obuf, out_hbm.at[pl.ds(base, _CH)], out_sems[b])

        @pl.when(g + _NBUF < n_ch)
        def _():
          pltpu.async_copy(
              x_hbm.at[pl.ds(base + _NBUF * _CH, _CH)], xbufs[b],
              in_sems[b])

      return carry

    lax.fori_loop(0, n_ch // _NBUF, chunk_group, 0)

    for b in range(_NBUF):
      g = n_ch - _NBUF + b
      pltpu.make_async_copy(
          obufs[b], out_hbm.at[pl.ds(base_w + g * _CH, _CH)],
          out_sems[b]).wait()

  return sc_gelu


def kernel(x, lut):
  n = x.size
  sc_gelu = _make_sc_call(n)
  out = sc_gelu(x.reshape(n), lut)
  return out.reshape(x.shape)
